# Initial kernel scaffold; baseline (speedup 1.0000x reference)
import jax
import jax.numpy as jnp
from jax.experimental import pallas as pl


def kernel(task_x, task_edge, res_x, res_edge, feedback, params):
    raise NotImplementedError("write your pallas kernel here")


# speedup vs baseline: 4.3072x; 4.3072x over previous
import functools
import numpy as np
import jax
import jax.numpy as jnp
from jax import lax
from jax.experimental import pallas as pl
from jax.experimental.pallas import tpu as pltpu

B = 8192
N_T = 10
N_R = 16
E = 64
D = 64
H = 64
N_PAIR = 45

_PAIR_I = tuple(int(i) for i in range(N_T) for j in range(i + 1, N_T))
_PAIR_J = tuple(int(j) for i in range(N_T) for j in range(i + 1, N_T))

BB = 64
EC = 8


def _hist(code, n_bins):
    iota = lax.broadcasted_iota(jnp.int32, (1, 1, n_bins), 2)
    acc = None
    for c in range(0, E, EC):
        part = (code[:, c:c + EC, None] == iota).astype(jnp.float32).sum(axis=1)
        acc = part if acc is None else acc + part
    return acc


def _layer_norm(z, g, b):
    m = z.mean(-1, keepdims=True)
    v = ((z - m) * (z - m)).mean(-1, keepdims=True)
    return (z - m) * lax.rsqrt(v + 1e-5) * g + b


def _agg(A3, scale, x3, n):
    outs = []
    for d in range(n):
        w = (A3[:, d, :, None] * x3).sum(axis=1) * scale[:, d:d + 1]
        outs.append(w[:, None, :])
    return jnp.concatenate(outs, axis=1)


def _gnn_block(x3, n, f_in, edge_ref,
               Wl1, bl1, Wr1, Wl2, bl2, Wr2, g1, b1, g2, b2, Wp, bp):
    src = edge_ref[:, 0, :]
    dst = edge_ref[:, 1, :]
    A = _hist(dst * n + src, n * n)
    A3 = A.reshape(BB, n, n)
    cnt = _hist(dst, n)
    scale = 1.0 / jnp.maximum(cnt, 1.0)

    xl = x3[:, :, 0:1] * Wl1[0:1, :]
    xr = x3[:, :, 0:1] * Wr1[0:1, :]
    for c in range(1, f_in):
        xl = xl + x3[:, :, c:c + 1] * Wl1[c:c + 1, :]
        xr = xr + x3[:, :, c:c + 1] * Wr1[c:c + 1, :]
    z1 = _agg(A3, scale, xl, n) + bl1 + xr
    h1 = jnp.maximum(_layer_norm(z1, g1, b1), 0.0)

    h1f = h1.reshape(BB * n, H)
    hl = jnp.dot(h1f, Wl2, preferred_element_type=jnp.float32).reshape(BB, n, H)
    hr = jnp.dot(h1f, Wr2, preferred_element_type=jnp.float32).reshape(BB, n, H)
    z2 = _agg(A3, scale, hl, n) + bl2 + hr
    h2 = jnp.maximum(_layer_norm(z2, g2, b2), 0.0)
    node = jnp.dot(h2.reshape(BB * n, H), Wp,
                   preferred_element_type=jnp.float32) + bp
    return node.reshape(BB, n, D)


def _fwd_kernel(tx_ref, te_ref, rx_ref, re_ref, fb_ref,
                tWl1, tbl1, tWr1, tWl2, tbl2, tWr2, tg1, tb1, tg2, tb2, tWp, tbp,
                rWl1, rbl1, rWr1, rWl2, rbl2, rWr2, rg1, rb1, rg2, rb2, rWp, rbp,
                Wq, Wk, Wv, Wo, bo,
                wnoop_t, wnoop_r, wnoop_rb, wnoop_fb, bnoop,
                wval_t, wval_r, wval_rb, wval_fb, bval,
                wsplit, bsplit, Wm1, bm1, wm2, bm2,
                logits_ref, value_ref):
    tx = tx_ref[...]
    rx = rx_ref[...]
    fb = fb_ref[...]

    task_node = _gnn_block(tx, N_T, 2, te_ref,
                           tWl1[...], tbl1[...], tWr1[...], tWl2[...], tbl2[...],
                           tWr2[...], tg1[...], tb1[...], tg2[...], tb2[...],
                           tWp[...], tbp[...])
    res_node = _gnn_block(rx, N_R, 3, re_ref,
                          rWl1[...], rbl1[...], rWr1[...], rWl2[...], rbl2[...],
                          rWr2[...], rg1[...], rb1[...], rg2[...], rb2[...],
                          rWp[...], rbp[...])

    mask = (jnp.abs(tx).sum(-1, keepdims=True) > 0).astype(jnp.float32)
    task_node = task_node * mask

    tnf = task_node.reshape(BB * N_T, D)
    rnf = res_node.reshape(BB * N_R, D)
    Q = jnp.dot(tnf, Wq[...], preferred_element_type=jnp.float32).reshape(BB, N_T, D)
    K = jnp.dot(rnf, Wk[...], preferred_element_type=jnp.float32).reshape(BB, N_R, D)
    V = jnp.dot(rnf, Wv[...], preferred_element_type=jnp.float32).reshape(BB, N_R, D)

    inv_sqrt_d = 1.0 / (D ** 0.5)
    P_list = []
    ctx_list = []
    for i in range(N_T):
        lg = (Q[:, i, None, :] * K).sum(-1) * inv_sqrt_d
        mx = lg.max(-1, keepdims=True)
        ex = jnp.exp(lg - mx)
        P = ex / ex.sum(-1, keepdims=True)
        P_list.append(P)
        ctx = (P[:, :, None] * V).sum(axis=1)
        ctx_list.append(ctx[:, None, :])
    ctx3 = jnp.concatenate(ctx_list, axis=1)
    te3 = task_node + jnp.dot(ctx3.reshape(BB * N_T, D), Wo[...],
                              preferred_element_type=jnp.float32
                              ).reshape(BB, N_T, D) + bo[...]

    n_real = jnp.maximum(mask.sum(axis=1), 1.0)
    tmean = te3.sum(axis=1) / n_real
    rmean = res_node.sum(axis=1) * (1.0 / N_R)
    rmin = rx.min(axis=1)
    rxmean = rx.sum(axis=1) * (1.0 / N_R)

    noop = ((tmean * wnoop_t[...]).sum(-1, keepdims=True)
            + (rmean * wnoop_r[...]).sum(-1, keepdims=True)
            + (rmin * wnoop_rb[...][:, 0:3]).sum(-1, keepdims=True)
            + (rxmean * wnoop_rb[...][:, 3:6]).sum(-1, keepdims=True)
            + (fb * wnoop_fb[...]).sum(-1, keepdims=True) + bnoop[...])
    val = ((tmean * wval_t[...]).sum(-1, keepdims=True)
           + (rmean * wval_r[...]).sum(-1, keepdims=True)
           + (rmin * wval_rb[...][:, 0:3]).sum(-1, keepdims=True)
           + (rxmean * wval_rb[...][:, 3:6]).sum(-1, keepdims=True)
           + (fb * wval_fb[...]).sum(-1, keepdims=True) + bval[...])
    value_ref[...] = val
    logits_ref[:, 0:1] = noop

    split = (te3 * wsplit[...][:, None, :]).sum(-1)
    logits_ref[:, 1:1 + N_T] = split + bsplit[...]

    W1 = Wm1[...]
    te_f = te3.reshape(BB * N_T, D)
    T1 = jnp.dot(te_f, W1[0:D, :], preferred_element_type=jnp.float32
                 ).reshape(BB, N_T, H)
    T2 = jnp.dot(te_f, W1[D:2 * D, :], preferred_element_type=jnp.float32
                 ).reshape(BB, N_T, H)
    wlq = W1[2 * D:2 * D + 1, :]
    bm1v = bm1[...]
    wm2v = wm2[...]
    bm2v = bm2[...]
    for p in range(N_PAIR):
        i = _PAIR_I[p]
        j = _PAIR_J[p]
        lq = (P_list[i] * P_list[j]).sum(-1, keepdims=True)
        m = jnp.maximum(T1[:, i, :] + T2[:, j, :] + lq * wlq + bm1v, 0.0)
        logits_ref[:, 11 + p:12 + p] = (m * wm2v).sum(-1, keepdims=True) + bm2v


def kernel(task_x, task_edge, res_x, res_edge, feedback, params):
    t = params['task']
    r = params['res']
    a = params['attn']
    h = params['heads']

    def row(v):
        return v.reshape(1, -1)

    wnoop = h['Wnoop']
    wval = h['Wval']
    weights = (
        t['Wl1'], row(t['bl1']), t['Wr1'], t['Wl2'], row(t['bl2']), t['Wr2'],
        row(t['g1']), row(t['b1']), row(t['g2']), row(t['b2']), t['Wp'], row(t['bp']),
        r['Wl1'], row(r['bl1']), r['Wr1'], r['Wl2'], row(r['bl2']), r['Wr2'],
        row(r['g1']), row(r['b1']), row(r['g2']), row(r['b2']), r['Wp'], row(r['bp']),
        a['Wq'], a['Wk'], a['Wv'], a['Wo'], row(a['bo']),
        row(wnoop[0:D, 0]), row(wnoop[D:2 * D, 0]), row(wnoop[2 * D:2 * D + 6, 0][0:6]),
        row(wnoop[2 * D + 6:2 * D + 12, 0]), row(h['bnoop']),
        row(wval[0:D, 0]), row(wval[D:2 * D, 0]), row(wval[2 * D:2 * D + 6, 0][0:6]),
        row(wval[2 * D + 6:2 * D + 12, 0]), row(h['bval']),
        row(h['Wsplit'][:, 0]), row(h['bsplit']),
        h['Wm1'], row(h['bm1']), row(h['Wm2'][:, 0]), row(h['bm2']),
    )

    grid = (B // BB,)

    def bspec(shape, blk):
        nd = len(shape)
        return pl.BlockSpec(blk, lambda i: (i,) + (0,) * (nd - 1))

    def wspec(w):
        nd = w.ndim
        return pl.BlockSpec(w.shape, lambda i, _n=nd: (0,) * _n)

    in_specs = [
        bspec((B, N_T, 2), (BB, N_T, 2)),
        bspec((B, 2, E), (BB, 2, E)),
        bspec((B, N_R, 3), (BB, N_R, 3)),
        bspec((B, 2, E), (BB, 2, E)),
        bspec((B, 6), (BB, 6)),
    ] + [wspec(w) for w in weights]

    out_shape = (
        jax.ShapeDtypeStruct((B, 56), jnp.float32),
        jax.ShapeDtypeStruct((B, 1), jnp.float32),
    )
    out_specs = (
        pl.BlockSpec((BB, 56), lambda i: (i, 0)),
        pl.BlockSpec((BB, 1), lambda i: (i, 0)),
    )

    logits, value = pl.pallas_call(
        _fwd_kernel,
        grid=grid,
        in_specs=in_specs,
        out_specs=out_specs,
        out_shape=out_shape,
        compiler_params=pltpu.CompilerParams(
            dimension_semantics=("arbitrary",),
        ),
    )(task_x, task_edge, res_x, res_edge, feedback, *weights)
    return (logits, value)


# speedup vs baseline: 11.1546x; 2.5898x over previous
import functools
import numpy as np
import jax
import jax.numpy as jnp
from jax import lax
from jax.experimental import pallas as pl
from jax.experimental.pallas import tpu as pltpu

B = 8192
N_T = 10
N_R = 16
E = 64
D = 64
H = 64
N_PAIR = 45

_PAIR_I = tuple(int(i) for i in range(N_T) for j in range(i + 1, N_T))
_PAIR_J = tuple(int(j) for i in range(N_T) for j in range(i + 1, N_T))

BB = 64
EC = 8


def _hist(code, n_bins):
    iota = lax.broadcasted_iota(jnp.int32, (1, 1, n_bins), 2)
    acc = None
    for c in range(0, E, EC):
        part = (code[:, c:c + EC, None] == iota).astype(jnp.float32).sum(axis=1)
        acc = part if acc is None else acc + part
    return acc


def _layer_norm(z, g, b):
    m = z.mean(-1, keepdims=True)
    v = ((z - m) * (z - m)).mean(-1, keepdims=True)
    return (z - m) * lax.rsqrt(v + 1e-5) * g + b


def _agg(An3, x3, n):
    acc = An3[:, :, 0:1] * x3[:, 0:1, :]
    for s in range(1, n):
        acc = acc + An3[:, :, s:s + 1] * x3[:, s:s + 1, :]
    return acc


def _gnn_block(x3, n, f_in, edge_ref,
               Wl1, bl1, Wr1, Wl2, bl2, Wr2, g1, b1, g2, b2, Wp, bp):
    src = edge_ref[:, 0, :]
    dst = edge_ref[:, 1, :]
    A = _hist(dst * n + src, n * n)
    A3 = A.reshape(BB, n, n)
    cnt3 = A3.sum(-1, keepdims=True)
    An3 = A3 * (1.0 / jnp.maximum(cnt3, 1.0))

    xl = x3[:, :, 0:1] * Wl1[0:1, :]
    xr = x3[:, :, 0:1] * Wr1[0:1, :]
    for c in range(1, f_in):
        xl = xl + x3[:, :, c:c + 1] * Wl1[c:c + 1, :]
        xr = xr + x3[:, :, c:c + 1] * Wr1[c:c + 1, :]
    z1 = _agg(An3, xl, n) + bl1 + xr
    h1 = jnp.maximum(_layer_norm(z1, g1, b1), 0.0)

    h1f = h1.reshape(BB * n, H)
    hl = jnp.dot(h1f, Wl2, preferred_element_type=jnp.float32).reshape(BB, n, H)
    hr = jnp.dot(h1f, Wr2, preferred_element_type=jnp.float32).reshape(BB, n, H)
    z2 = _agg(An3, hl, n) + bl2 + hr
    h2 = jnp.maximum(_layer_norm(z2, g2, b2), 0.0)
    node = jnp.dot(h2.reshape(BB * n, H), Wp,
                   preferred_element_type=jnp.float32) + bp
    return node.reshape(BB, n, D)


def _fwd_kernel(tx_ref, te_ref, rx_ref, re_ref, fb_ref,
                tWl1, tbl1, tWr1, tWl2, tbl2, tWr2, tg1, tb1, tg2, tb2, tWp, tbp,
                rWl1, rbl1, rWr1, rWl2, rbl2, rWr2, rg1, rb1, rg2, rb2, rWp, rbp,
                Wq, Wk, Wv, Wo, bo,
                wnoop_t, wnoop_r, wnoop_rb, wnoop_fb, bnoop,
                wval_t, wval_r, wval_rb, wval_fb, bval,
                wsplit, bsplit, Wm1, bm1, wm2, bm2,
                logits_ref, value_ref):
    tx = tx_ref[...]
    rx = rx_ref[...]
    fb = fb_ref[...]

    task_node = _gnn_block(tx, N_T, 2, te_ref,
                           tWl1[...], tbl1[...], tWr1[...], tWl2[...], tbl2[...],
                           tWr2[...], tg1[...], tb1[...], tg2[...], tb2[...],
                           tWp[...], tbp[...])
    res_node = _gnn_block(rx, N_R, 3, re_ref,
                          rWl1[...], rbl1[...], rWr1[...], rWl2[...], rbl2[...],
                          rWr2[...], rg1[...], rb1[...], rg2[...], rb2[...],
                          rWp[...], rbp[...])

    mask = (jnp.abs(tx).sum(-1, keepdims=True) > 0).astype(jnp.float32)
    task_node = task_node * mask

    tnf = task_node.reshape(BB * N_T, D)
    rnf = res_node.reshape(BB * N_R, D)
    Q = jnp.dot(tnf, Wq[...], preferred_element_type=jnp.float32).reshape(BB, N_T, D)
    K = jnp.dot(rnf, Wk[...], preferred_element_type=jnp.float32).reshape(BB, N_R, D)
    V = jnp.dot(rnf, Wv[...], preferred_element_type=jnp.float32).reshape(BB, N_R, D)

    inv_sqrt_d = 1.0 / (D ** 0.5)
    lg_rows = []
    for i in range(N_T):
        prod = Q[:, i:i + 1, :] * K
        lg_rows.append(prod.sum(-1)[:, None, :])
    lg3 = jnp.concatenate(lg_rows, axis=1) * inv_sqrt_d
    mx = lg3.max(-1, keepdims=True)
    ex = jnp.exp(lg3 - mx)
    P3 = ex / ex.sum(-1, keepdims=True)
    ctx3 = P3[:, :, 0:1] * V[:, 0:1, :]
    for j in range(1, N_R):
        ctx3 = ctx3 + P3[:, :, j:j + 1] * V[:, j:j + 1, :]
    te3 = task_node + jnp.dot(ctx3.reshape(BB * N_T, D), Wo[...],
                              preferred_element_type=jnp.float32
                              ).reshape(BB, N_T, D) + bo[...]

    n_real = jnp.maximum(mask.sum(axis=1), 1.0)
    tmean = te3.sum(axis=1) / n_real
    rmean = res_node.sum(axis=1) * (1.0 / N_R)
    rmin = rx.min(axis=1)
    rxmean = rx.sum(axis=1) * (1.0 / N_R)

    noop = ((tmean * wnoop_t[...]).sum(-1, keepdims=True)
            + (rmean * wnoop_r[...]).sum(-1, keepdims=True)
            + (rmin * wnoop_rb[...][:, 0:3]).sum(-1, keepdims=True)
            + (rxmean * wnoop_rb[...][:, 3:6]).sum(-1, keepdims=True)
            + (fb * wnoop_fb[...]).sum(-1, keepdims=True) + bnoop[...])
    val = ((tmean * wval_t[...]).sum(-1, keepdims=True)
           + (rmean * wval_r[...]).sum(-1, keepdims=True)
           + (rmin * wval_rb[...][:, 0:3]).sum(-1, keepdims=True)
           + (rxmean * wval_rb[...][:, 3:6]).sum(-1, keepdims=True)
           + (fb * wval_fb[...]).sum(-1, keepdims=True) + bval[...])
    value_ref[...] = val
    logits_ref[:, 0:1] = noop

    split = (te3 * wsplit[...][:, None, :]).sum(-1)
    logits_ref[:, 1:1 + N_T] = split + bsplit[...]

    W1 = Wm1[...]
    te_f = te3.reshape(BB * N_T, D)
    T1 = jnp.dot(te_f, W1[0:D, :], preferred_element_type=jnp.float32
                 ).reshape(BB, N_T, H)
    T2 = jnp.dot(te_f, W1[D:2 * D, :], preferred_element_type=jnp.float32
                 ).reshape(BB, N_T, H)
    wlq = W1[2 * D:2 * D + 1, :]
    Pi3 = jnp.concatenate([P3[:, i:i + 1, :] for i in _PAIR_I], axis=1)
    Pj3 = jnp.concatenate([P3[:, j:j + 1, :] for j in _PAIR_J], axis=1)
    lq3 = (Pi3 * Pj3).sum(-1, keepdims=True)
    T1p = jnp.concatenate([T1[:, i:i + 1, :] for i in _PAIR_I], axis=1)
    T2p = jnp.concatenate([T2[:, j:j + 1, :] for j in _PAIR_J], axis=1)
    m3 = jnp.maximum(T1p + T2p + lq3 * wlq + bm1[...], 0.0)
    merge = (m3 * wm2[...]).sum(-1)
    logits_ref[:, 11:11 + N_PAIR] = merge + bm2[...]


def kernel(task_x, task_edge, res_x, res_edge, feedback, params):
    t = params['task']
    r = params['res']
    a = params['attn']
    h = params['heads']

    def row(v):
        return v.reshape(1, -1)

    wnoop = h['Wnoop']
    wval = h['Wval']
    weights = (
        t['Wl1'], row(t['bl1']), t['Wr1'], t['Wl2'], row(t['bl2']), t['Wr2'],
        row(t['g1']), row(t['b1']), row(t['g2']), row(t['b2']), t['Wp'], row(t['bp']),
        r['Wl1'], row(r['bl1']), r['Wr1'], r['Wl2'], row(r['bl2']), r['Wr2'],
        row(r['g1']), row(r['b1']), row(r['g2']), row(r['b2']), r['Wp'], row(r['bp']),
        a['Wq'], a['Wk'], a['Wv'], a['Wo'], row(a['bo']),
        row(wnoop[0:D, 0]), row(wnoop[D:2 * D, 0]), row(wnoop[2 * D:2 * D + 6, 0][0:6]),
        row(wnoop[2 * D + 6:2 * D + 12, 0]), row(h['bnoop']),
        row(wval[0:D, 0]), row(wval[D:2 * D, 0]), row(wval[2 * D:2 * D + 6, 0][0:6]),
        row(wval[2 * D + 6:2 * D + 12, 0]), row(h['bval']),
        row(h['Wsplit'][:, 0]), row(h['bsplit']),
        h['Wm1'], row(h['bm1']), row(h['Wm2'][:, 0]), row(h['bm2']),
    )

    grid = (B // BB,)

    def bspec(shape, blk):
        nd = len(shape)
        return pl.BlockSpec(blk, lambda i: (i,) + (0,) * (nd - 1))

    def wspec(w):
        nd = w.ndim
        return pl.BlockSpec(w.shape, lambda i, _n=nd: (0,) * _n)

    in_specs = [
        bspec((B, N_T, 2), (BB, N_T, 2)),
        bspec((B, 2, E), (BB, 2, E)),
        bspec((B, N_R, 3), (BB, N_R, 3)),
        bspec((B, 2, E), (BB, 2, E)),
        bspec((B, 6), (BB, 6)),
    ] + [wspec(w) for w in weights]

    out_shape = (
        jax.ShapeDtypeStruct((B, 56), jnp.float32),
        jax.ShapeDtypeStruct((B, 1), jnp.float32),
    )
    out_specs = (
        pl.BlockSpec((BB, 56), lambda i: (i, 0)),
        pl.BlockSpec((BB, 1), lambda i: (i, 0)),
    )

    logits, value = pl.pallas_call(
        _fwd_kernel,
        grid=grid,
        in_specs=in_specs,
        out_specs=out_specs,
        out_shape=out_shape,
        compiler_params=pltpu.CompilerParams(
            dimension_semantics=("arbitrary",),
        ),
    )(task_x, task_edge, res_x, res_edge, feedback, *weights)
    return (logits, value)


# speedup vs baseline: 12.6944x; 1.1380x over previous
import functools
import numpy as np
import jax
import jax.numpy as jnp
from jax import lax
from jax.experimental import pallas as pl
from jax.experimental.pallas import tpu as pltpu
from jax.experimental.pallas import tpu_sc as plsc

B = 8192
N_T = 10
N_R = 16
E = 64
D = 64
H = 64
N_PAIR = 45

_PAIR_I = tuple(int(i) for i in range(N_T) for j in range(i + 1, N_T))
_PAIR_J = tuple(int(j) for i in range(N_T) for j in range(i + 1, N_T))

NT2 = N_T * N_T
NR2 = N_R * N_R
BB = 64
NW = 32
GPW = B // NW
CH = 16


NGRP = B // CH
GRP_PW = NGRP // NW


def _sc_hist(task_edge2, res_edge2):
    tt = task_edge2.reshape(NGRP, CH, 2 * E).transpose(0, 2, 1)
    rt = res_edge2.reshape(NGRP, CH, 2 * E).transpose(0, 2, 1)
    mesh = plsc.VectorSubcoreMesh(core_axis_name="c", subcore_axis_name="s")

    @functools.partial(
        pl.kernel,
        mesh=mesh,
        out_type=(
            jax.ShapeDtypeStruct((B * NT2,), jnp.float32),
            jax.ShapeDtypeStruct((B * NR2,), jnp.float32),
        ),
        scratch_types=[
            pltpu.VMEM((2 * E, CH), jnp.int32),
            pltpu.VMEM((2 * E, CH), jnp.int32),
            pltpu.VMEM((CH * NT2,), jnp.float32),
            pltpu.VMEM((CH * NR2,), jnp.float32),
        ],
        compiler_params=pltpu.CompilerParams(needs_layout_passes=False),
    )
    def k(te_hbm, re_hbm, at_hbm, ar_hbm, tev, rev, acct, accr):
        wid = lax.axis_index("s") * 2 + lax.axis_index("c")
        ones = jnp.full((16,), 1.0, jnp.float32)
        zeros = jnp.zeros((16,), jnp.float32)
        lane = lax.iota(jnp.int32, 16)

        def chunk(ci, carry):
            grp = wid * GRP_PW + ci
            pltpu.sync_copy(te_hbm.at[grp], tev)
            pltpu.sync_copy(re_hbm.at[grp], rev)
            for z in range(CH * NT2 // 16):
                acct[pl.ds(z * 16, 16)] = zeros
            for z in range(CH * NR2 // 16):
                accr[pl.ds(z * 16, 16)] = zeros
            for e in range(E):
                st = tev[e]
                dt = tev[e + E]
                plsc.addupdate_scatter(acct, [lane * NT2 + dt * N_T + st], ones)
                sr = rev[e]
                dr = rev[e + E]
                plsc.addupdate_scatter(accr, [lane * NR2 + dr * N_R + sr], ones)
            pltpu.sync_copy(acct, at_hbm.at[pl.ds(grp * CH * NT2, CH * NT2)])
            pltpu.sync_copy(accr, ar_hbm.at[pl.ds(grp * CH * NR2, CH * NR2)])
            return carry

        lax.fori_loop(0, GRP_PW, chunk, 0)

    return k(tt, rt)


def _layer_norm(z, g, b):
    m = z.mean(-1, keepdims=True)
    v = ((z - m) * (z - m)).mean(-1, keepdims=True)
    return (z - m) * lax.rsqrt(v + 1e-5) * g + b


def _tree_sum(terms):
    while len(terms) > 1:
        nxt = [terms[i] + terms[i + 1] for i in range(0, len(terms) - 1, 2)]
        if len(terms) % 2:
            nxt.append(terms[-1])
        terms = nxt
    return terms[0]


def _agg(An3, x3, n):
    return _tree_sum([An3[:, :, s:s + 1] * x3[:, s:s + 1, :]
                      for s in range(n)])


def _gnn_block(x3, n, f_in, A_ref,
               Wl1, bl1, Wr1, Wl2, bl2, Wr2, g1, b1, g2, b2, Wp, bp):
    A3 = A_ref[...].reshape(BB, n, n)
    cnt3 = A3.sum(-1, keepdims=True)
    An3 = A3 * (1.0 / jnp.maximum(cnt3, 1.0))

    xl = x3[:, :, 0:1] * Wl1[0:1, :]
    xr = x3[:, :, 0:1] * Wr1[0:1, :]
    for c in range(1, f_in):
        xl = xl + x3[:, :, c:c + 1] * Wl1[c:c + 1, :]
        xr = xr + x3[:, :, c:c + 1] * Wr1[c:c + 1, :]
    z1 = _agg(An3, xl, n) + bl1 + xr
    h1 = jnp.maximum(_layer_norm(z1, g1, b1), 0.0)

    h1f = h1.reshape(BB * n, H)
    hl = jnp.dot(h1f, Wl2, preferred_element_type=jnp.float32).reshape(BB, n, H)
    hr = jnp.dot(h1f, Wr2, preferred_element_type=jnp.float32).reshape(BB, n, H)
    z2 = _agg(An3, hl, n) + bl2 + hr
    h2 = jnp.maximum(_layer_norm(z2, g2, b2), 0.0)
    node = jnp.dot(h2.reshape(BB * n, H), Wp,
                   preferred_element_type=jnp.float32) + bp
    return node.reshape(BB, n, D)


def _fwd_kernel(tx_ref, at_ref, rx_ref, ar_ref, fb_ref,
                tWl1, tbl1, tWr1, tWl2, tbl2, tWr2, tg1, tb1, tg2, tb2, tWp, tbp,
                rWl1, rbl1, rWr1, rWl2, rbl2, rWr2, rg1, rb1, rg2, rb2, rWp, rbp,
                Wq, Wk, Wv, Wo, bo,
                wnoop_t, wnoop_r, wnoop_rb, wnoop_fb, bnoop,
                wval_t, wval_r, wval_rb, wval_fb, bval,
                wsplit, bsplit, Wm1, bm1, wm2, bm2,
                logits_ref, value_ref):
    tx = tx_ref[...]
    rx = rx_ref[...]
    fb = fb_ref[...]

    task_node = _gnn_block(tx, N_T, 2, at_ref,
                           tWl1[...], tbl1[...], tWr1[...], tWl2[...], tbl2[...],
                           tWr2[...], tg1[...], tb1[...], tg2[...], tb2[...],
                           tWp[...], tbp[...])
    res_node = _gnn_block(rx, N_R, 3, ar_ref,
                          rWl1[...], rbl1[...], rWr1[...], rWl2[...], rbl2[...],
                          rWr2[...], rg1[...], rb1[...], rg2[...], rb2[...],
                          rWp[...], rbp[...])

    mask = (jnp.abs(tx).sum(-1, keepdims=True) > 0).astype(jnp.float32)
    task_node = task_node * mask

    tnf = task_node.reshape(BB * N_T, D)
    rnf = res_node.reshape(BB * N_R, D)
    Q = jnp.dot(tnf, Wq[...], preferred_element_type=jnp.float32).reshape(BB, N_T, D)
    K = jnp.dot(rnf, Wk[...], preferred_element_type=jnp.float32).reshape(BB, N_R, D)
    V = jnp.dot(rnf, Wv[...], preferred_element_type=jnp.float32).reshape(BB, N_R, D)

    inv_sqrt_d = 1.0 / (D ** 0.5)
    lg_rows = []
    for i in range(N_T):
        prod = Q[:, i:i + 1, :] * K
        lg_rows.append(prod.sum(-1)[:, None, :])
    lg3 = jnp.concatenate(lg_rows, axis=1) * inv_sqrt_d
    mx = lg3.max(-1, keepdims=True)
    ex = jnp.exp(lg3 - mx)
    P3 = ex / ex.sum(-1, keepdims=True)
    ctx3 = _tree_sum([P3[:, :, j:j + 1] * V[:, j:j + 1, :]
                      for j in range(N_R)])
    te3 = task_node + jnp.dot(ctx3.reshape(BB * N_T, D), Wo[...],
                              preferred_element_type=jnp.float32
                              ).reshape(BB, N_T, D) + bo[...]

    n_real = jnp.maximum(mask.sum(axis=1), 1.0)
    tmean = te3.sum(axis=1) / n_real
    rmean = res_node.sum(axis=1) * (1.0 / N_R)
    rmin = rx.min(axis=1)
    rxmean = rx.sum(axis=1) * (1.0 / N_R)

    noop = ((tmean * wnoop_t[...]).sum(-1, keepdims=True)
            + (rmean * wnoop_r[...]).sum(-1, keepdims=True)
            + (rmin * wnoop_rb[...][:, 0:3]).sum(-1, keepdims=True)
            + (rxmean * wnoop_rb[...][:, 3:6]).sum(-1, keepdims=True)
            + (fb * wnoop_fb[...]).sum(-1, keepdims=True) + bnoop[...])
    val = ((tmean * wval_t[...]).sum(-1, keepdims=True)
           + (rmean * wval_r[...]).sum(-1, keepdims=True)
           + (rmin * wval_rb[...][:, 0:3]).sum(-1, keepdims=True)
           + (rxmean * wval_rb[...][:, 3:6]).sum(-1, keepdims=True)
           + (fb * wval_fb[...]).sum(-1, keepdims=True) + bval[...])
    value_ref[...] = val
    logits_ref[:, 0:1] = noop

    split = (te3 * wsplit[...][:, None, :]).sum(-1)
    logits_ref[:, 1:1 + N_T] = split + bsplit[...]

    W1 = Wm1[...]
    te_f = te3.reshape(BB * N_T, D)
    T1 = jnp.dot(te_f, W1[0:D, :], preferred_element_type=jnp.float32
                 ).reshape(BB, N_T, H)
    T2 = jnp.dot(te_f, W1[D:2 * D, :], preferred_element_type=jnp.float32
                 ).reshape(BB, N_T, H)
    wlq = W1[2 * D:2 * D + 1, :]
    Pi3 = jnp.concatenate([P3[:, i:i + 1, :] for i in _PAIR_I], axis=1)
    Pj3 = jnp.concatenate([P3[:, j:j + 1, :] for j in _PAIR_J], axis=1)
    lq3 = (Pi3 * Pj3).sum(-1, keepdims=True)
    T1p = jnp.concatenate([T1[:, i:i + 1, :] for i in _PAIR_I], axis=1)
    T2p = jnp.concatenate([T2[:, j:j + 1, :] for j in _PAIR_J], axis=1)
    m3 = jnp.maximum(T1p + T2p + lq3 * wlq + bm1[...], 0.0)
    merge = (m3 * wm2[...]).sum(-1)
    logits_ref[:, 11:11 + N_PAIR] = merge + bm2[...]


def kernel(task_x, task_edge, res_x, res_edge, feedback, params):
    t = params['task']
    r = params['res']
    a = params['attn']
    h = params['heads']

    A_t, A_r = _sc_hist(task_edge.reshape(B, 2 * E), res_edge.reshape(B, 2 * E))
    A_t = A_t.reshape(B, NT2)
    A_r = A_r.reshape(B, NR2)

    def row(v):
        return v.reshape(1, -1)

    wnoop = h['Wnoop']
    wval = h['Wval']
    weights = (
        t['Wl1'], row(t['bl1']), t['Wr1'], t['Wl2'], row(t['bl2']), t['Wr2'],
        row(t['g1']), row(t['b1']), row(t['g2']), row(t['b2']), t['Wp'], row(t['bp']),
        r['Wl1'], row(r['bl1']), r['Wr1'], r['Wl2'], row(r['bl2']), r['Wr2'],
        row(r['g1']), row(r['b1']), row(r['g2']), row(r['b2']), r['Wp'], row(r['bp']),
        a['Wq'], a['Wk'], a['Wv'], a['Wo'], row(a['bo']),
        row(wnoop[0:D, 0]), row(wnoop[D:2 * D, 0]), row(wnoop[2 * D:2 * D + 6, 0][0:6]),
        row(wnoop[2 * D + 6:2 * D + 12, 0]), row(h['bnoop']),
        row(wval[0:D, 0]), row(wval[D:2 * D, 0]), row(wval[2 * D:2 * D + 6, 0][0:6]),
        row(wval[2 * D + 6:2 * D + 12, 0]), row(h['bval']),
        row(h['Wsplit'][:, 0]), row(h['bsplit']),
        h['Wm1'], row(h['bm1']), row(h['Wm2'][:, 0]), row(h['bm2']),
    )

    grid = (B // BB,)

    def bspec(shape, blk):
        nd = len(shape)
        return pl.BlockSpec(blk, lambda i: (i,) + (0,) * (nd - 1))

    def wspec(w):
        nd = w.ndim
        return pl.BlockSpec(w.shape, lambda i, _n=nd: (0,) * _n)

    in_specs = [
        bspec((B, N_T, 2), (BB, N_T, 2)),
        bspec((B, NT2), (BB, NT2)),
        bspec((B, N_R, 3), (BB, N_R, 3)),
        bspec((B, NR2), (BB, NR2)),
        bspec((B, 6), (BB, 6)),
    ] + [wspec(w) for w in weights]

    out_shape = (
        jax.ShapeDtypeStruct((B, 56), jnp.float32),
        jax.ShapeDtypeStruct((B, 1), jnp.float32),
    )
    out_specs = (
        pl.BlockSpec((BB, 56), lambda i: (i, 0)),
        pl.BlockSpec((BB, 1), lambda i: (i, 0)),
    )

    logits, value = pl.pallas_call(
        _fwd_kernel,
        grid=grid,
        in_specs=in_specs,
        out_specs=out_specs,
        out_shape=out_shape,
        compiler_params=pltpu.CompilerParams(
            dimension_semantics=("arbitrary",),
        ),
    )(task_x, A_t, res_x, A_r, feedback, *weights)
    return (logits, value)


# speedup vs baseline: 23.6525x; 1.8632x over previous
import functools
import numpy as np
import jax
import jax.numpy as jnp
from jax import lax
from jax.experimental import pallas as pl
from jax.experimental.pallas import tpu as pltpu
from jax.experimental.pallas import tpu_sc as plsc

B = 8192
N_T = 10
N_R = 16
E = 64
D = 64
H = 64
N_PAIR = 45

_PAIR_I = tuple(int(i) for i in range(N_T) for j in range(i + 1, N_T))
_PAIR_J = tuple(int(j) for i in range(N_T) for j in range(i + 1, N_T))

NT2 = N_T * N_T
NR2 = N_R * N_R
BB = 64
NW = 32
GPW = B // NW
CH = 16


NGRP = B // CH
GRP_PW = NGRP // NW


def _sc_hist(task_edge2, res_edge2):
    tt = task_edge2.reshape(NGRP, CH, 2 * E).transpose(0, 2, 1)
    rt = res_edge2.reshape(NGRP, CH, 2 * E).transpose(0, 2, 1)
    mesh = plsc.VectorSubcoreMesh(core_axis_name="c", subcore_axis_name="s")

    @functools.partial(
        pl.kernel,
        mesh=mesh,
        out_type=(
            jax.ShapeDtypeStruct((B * NT2,), jnp.float32),
            jax.ShapeDtypeStruct((B * NR2,), jnp.float32),
        ),
        scratch_types=[
            pltpu.VMEM((2 * E, CH), jnp.int32),
            pltpu.VMEM((2 * E, CH), jnp.int32),
            pltpu.VMEM((CH * NT2,), jnp.float32),
            pltpu.VMEM((CH * NR2,), jnp.float32),
        ],
        compiler_params=pltpu.CompilerParams(needs_layout_passes=False),
    )
    def k(te_hbm, re_hbm, at_hbm, ar_hbm, tev, rev, acct, accr):
        wid = lax.axis_index("s") * 2 + lax.axis_index("c")
        ones = jnp.full((16,), 1.0, jnp.float32)
        zeros = jnp.zeros((16,), jnp.float32)
        lane = lax.iota(jnp.int32, 16)

        def chunk(ci, carry):
            grp = wid * GRP_PW + ci
            pltpu.sync_copy(te_hbm.at[grp], tev)
            pltpu.sync_copy(re_hbm.at[grp], rev)
            for z in range(CH * NT2 // 16):
                acct[pl.ds(z * 16, 16)] = zeros
            for z in range(CH * NR2 // 16):
                accr[pl.ds(z * 16, 16)] = zeros
            for e in range(E):
                st = tev[e]
                dt = tev[e + E]
                plsc.addupdate_scatter(acct, [lane * NT2 + dt * N_T + st], ones)
                sr = rev[e]
                dr = rev[e + E]
                plsc.addupdate_scatter(accr, [lane * NR2 + dr * N_R + sr], ones)
            pltpu.sync_copy(acct, at_hbm.at[pl.ds(grp * CH * NT2, CH * NT2)])
            pltpu.sync_copy(accr, ar_hbm.at[pl.ds(grp * CH * NR2, CH * NR2)])
            return carry

        lax.fori_loop(0, GRP_PW, chunk, 0)

    return k(tt, rt)


def _layer_norm(z, g, b):
    m = z.mean(-1, keepdims=True)
    v = ((z - m) * (z - m)).mean(-1, keepdims=True)
    return (z - m) * lax.rsqrt(v + 1e-5) * g + b


def _tree_sum(terms):
    while len(terms) > 1:
        nxt = [terms[i] + terms[i + 1] for i in range(0, len(terms) - 1, 2)]
        if len(terms) % 2:
            nxt.append(terms[-1])
        terms = nxt
    return terms[0]


def _agg(An3, x3, n):
    return jax.lax.dot_general(An3, x3, (((2,), (1,)), ((0,), (0,))),
                               preferred_element_type=jnp.float32)


def _gnn_block(x3, n, f_in, A_ref,
               Wl1, bl1, Wr1, Wl2, bl2, Wr2, g1, b1, g2, b2, Wp, bp):
    A3 = A_ref[...].reshape(BB, n, n)
    cnt3 = A3.sum(-1, keepdims=True)
    An3 = A3 * (1.0 / jnp.maximum(cnt3, 1.0))

    xl = x3[:, :, 0:1] * Wl1[0:1, :]
    xr = x3[:, :, 0:1] * Wr1[0:1, :]
    for c in range(1, f_in):
        xl = xl + x3[:, :, c:c + 1] * Wl1[c:c + 1, :]
        xr = xr + x3[:, :, c:c + 1] * Wr1[c:c + 1, :]
    z1 = _agg(An3, xl, n) + bl1 + xr
    h1 = jnp.maximum(_layer_norm(z1, g1, b1), 0.0)

    h1f = h1.reshape(BB * n, H)
    hl = jnp.dot(h1f, Wl2, preferred_element_type=jnp.float32).reshape(BB, n, H)
    hr = jnp.dot(h1f, Wr2, preferred_element_type=jnp.float32).reshape(BB, n, H)
    z2 = _agg(An3, hl, n) + bl2 + hr
    h2 = jnp.maximum(_layer_norm(z2, g2, b2), 0.0)
    node = jnp.dot(h2.reshape(BB * n, H), Wp,
                   preferred_element_type=jnp.float32) + bp
    return node.reshape(BB, n, D)


def _fwd_kernel(tx_ref, at_ref, rx_ref, ar_ref, fb_ref,
                tWl1, tbl1, tWr1, tWl2, tbl2, tWr2, tg1, tb1, tg2, tb2, tWp, tbp,
                rWl1, rbl1, rWr1, rWl2, rbl2, rWr2, rg1, rb1, rg2, rb2, rWp, rbp,
                Wq, Wk, Wv, Wo, bo,
                wnoop_t, wnoop_r, wnoop_rb, wnoop_fb, bnoop,
                wval_t, wval_r, wval_rb, wval_fb, bval,
                wsplit, bsplit, Wm1, bm1, wm2, bm2,
                logits_ref, value_ref):
    tx = tx_ref[...]
    rx = rx_ref[...]
    fb = fb_ref[...]

    task_node = _gnn_block(tx, N_T, 2, at_ref,
                           tWl1[...], tbl1[...], tWr1[...], tWl2[...], tbl2[...],
                           tWr2[...], tg1[...], tb1[...], tg2[...], tb2[...],
                           tWp[...], tbp[...])
    res_node = _gnn_block(rx, N_R, 3, ar_ref,
                          rWl1[...], rbl1[...], rWr1[...], rWl2[...], rbl2[...],
                          rWr2[...], rg1[...], rb1[...], rg2[...], rb2[...],
                          rWp[...], rbp[...])

    mask = (jnp.abs(tx).sum(-1, keepdims=True) > 0).astype(jnp.float32)
    task_node = task_node * mask

    tnf = task_node.reshape(BB * N_T, D)
    rnf = res_node.reshape(BB * N_R, D)
    Q = jnp.dot(tnf, Wq[...], preferred_element_type=jnp.float32).reshape(BB, N_T, D)
    K = jnp.dot(rnf, Wk[...], preferred_element_type=jnp.float32).reshape(BB, N_R, D)
    V = jnp.dot(rnf, Wv[...], preferred_element_type=jnp.float32).reshape(BB, N_R, D)

    inv_sqrt_d = 1.0 / (D ** 0.5)
    lg3 = jax.lax.dot_general(Q, K, (((2,), (2,)), ((0,), (0,))),
                              preferred_element_type=jnp.float32) * inv_sqrt_d
    mx = lg3.max(-1, keepdims=True)
    ex = jnp.exp(lg3 - mx)
    P3 = ex / ex.sum(-1, keepdims=True)
    ctx3 = jax.lax.dot_general(P3, V, (((2,), (1,)), ((0,), (0,))),
                               preferred_element_type=jnp.float32)
    te3 = task_node + jnp.dot(ctx3.reshape(BB * N_T, D), Wo[...],
                              preferred_element_type=jnp.float32
                              ).reshape(BB, N_T, D) + bo[...]

    n_real = jnp.maximum(mask.sum(axis=1), 1.0)
    tmean = te3.sum(axis=1) / n_real
    rmean = res_node.sum(axis=1) * (1.0 / N_R)
    rmin = rx.min(axis=1)
    rxmean = rx.sum(axis=1) * (1.0 / N_R)

    noop = ((tmean * wnoop_t[...]).sum(-1, keepdims=True)
            + (rmean * wnoop_r[...]).sum(-1, keepdims=True)
            + (rmin * wnoop_rb[...][:, 0:3]).sum(-1, keepdims=True)
            + (rxmean * wnoop_rb[...][:, 3:6]).sum(-1, keepdims=True)
            + (fb * wnoop_fb[...]).sum(-1, keepdims=True) + bnoop[...])
    val = ((tmean * wval_t[...]).sum(-1, keepdims=True)
           + (rmean * wval_r[...]).sum(-1, keepdims=True)
           + (rmin * wval_rb[...][:, 0:3]).sum(-1, keepdims=True)
           + (rxmean * wval_rb[...][:, 3:6]).sum(-1, keepdims=True)
           + (fb * wval_fb[...]).sum(-1, keepdims=True) + bval[...])
    value_ref[...] = val
    logits_ref[:, 0:1] = noop

    split = (te3 * wsplit[...][:, None, :]).sum(-1)
    logits_ref[:, 1:1 + N_T] = split + bsplit[...]

    W1 = Wm1[...]
    te_f = te3.reshape(BB * N_T, D)
    T1 = jnp.dot(te_f, W1[0:D, :], preferred_element_type=jnp.float32
                 ).reshape(BB, N_T, H)
    T2 = jnp.dot(te_f, W1[D:2 * D, :], preferred_element_type=jnp.float32
                 ).reshape(BB, N_T, H)
    wlq = W1[2 * D:2 * D + 1, :]
    Pi3 = jnp.concatenate([P3[:, i:i + 1, :] for i in _PAIR_I], axis=1)
    Pj3 = jnp.concatenate([P3[:, j:j + 1, :] for j in _PAIR_J], axis=1)
    lq3 = (Pi3 * Pj3).sum(-1, keepdims=True)
    T1p = jnp.concatenate([T1[:, i:i + 1, :] for i in _PAIR_I], axis=1)
    T2p = jnp.concatenate([T2[:, j:j + 1, :] for j in _PAIR_J], axis=1)
    m3 = jnp.maximum(T1p + T2p + lq3 * wlq + bm1[...], 0.0)
    merge = (m3 * wm2[...]).sum(-1)
    logits_ref[:, 11:11 + N_PAIR] = merge + bm2[...]


def kernel(task_x, task_edge, res_x, res_edge, feedback, params):
    t = params['task']
    r = params['res']
    a = params['attn']
    h = params['heads']

    A_t, A_r = _sc_hist(task_edge.reshape(B, 2 * E), res_edge.reshape(B, 2 * E))
    A_t = A_t.reshape(B, NT2)
    A_r = A_r.reshape(B, NR2)

    def row(v):
        return v.reshape(1, -1)

    wnoop = h['Wnoop']
    wval = h['Wval']
    weights = (
        t['Wl1'], row(t['bl1']), t['Wr1'], t['Wl2'], row(t['bl2']), t['Wr2'],
        row(t['g1']), row(t['b1']), row(t['g2']), row(t['b2']), t['Wp'], row(t['bp']),
        r['Wl1'], row(r['bl1']), r['Wr1'], r['Wl2'], row(r['bl2']), r['Wr2'],
        row(r['g1']), row(r['b1']), row(r['g2']), row(r['b2']), r['Wp'], row(r['bp']),
        a['Wq'], a['Wk'], a['Wv'], a['Wo'], row(a['bo']),
        row(wnoop[0:D, 0]), row(wnoop[D:2 * D, 0]), row(wnoop[2 * D:2 * D + 6, 0][0:6]),
        row(wnoop[2 * D + 6:2 * D + 12, 0]), row(h['bnoop']),
        row(wval[0:D, 0]), row(wval[D:2 * D, 0]), row(wval[2 * D:2 * D + 6, 0][0:6]),
        row(wval[2 * D + 6:2 * D + 12, 0]), row(h['bval']),
        row(h['Wsplit'][:, 0]), row(h['bsplit']),
        h['Wm1'], row(h['bm1']), row(h['Wm2'][:, 0]), row(h['bm2']),
    )

    grid = (B // BB,)

    def bspec(shape, blk):
        nd = len(shape)
        return pl.BlockSpec(blk, lambda i: (i,) + (0,) * (nd - 1))

    def wspec(w):
        nd = w.ndim
        return pl.BlockSpec(w.shape, lambda i, _n=nd: (0,) * _n)

    in_specs = [
        bspec((B, N_T, 2), (BB, N_T, 2)),
        bspec((B, NT2), (BB, NT2)),
        bspec((B, N_R, 3), (BB, N_R, 3)),
        bspec((B, NR2), (BB, NR2)),
        bspec((B, 6), (BB, 6)),
    ] + [wspec(w) for w in weights]

    out_shape = (
        jax.ShapeDtypeStruct((B, 56), jnp.float32),
        jax.ShapeDtypeStruct((B, 1), jnp.float32),
    )
    out_specs = (
        pl.BlockSpec((BB, 56), lambda i: (i, 0)),
        pl.BlockSpec((BB, 1), lambda i: (i, 0)),
    )

    logits, value = pl.pallas_call(
        _fwd_kernel,
        grid=grid,
        in_specs=in_specs,
        out_specs=out_specs,
        out_shape=out_shape,
        compiler_params=pltpu.CompilerParams(
            dimension_semantics=("arbitrary",),
        ),
    )(task_x, A_t, res_x, A_r, feedback, *weights)
    return (logits, value)


# speedup vs baseline: 27.0983x; 1.1457x over previous
import functools
import numpy as np
import jax
import jax.numpy as jnp
from jax import lax
from jax.experimental import pallas as pl
from jax.experimental.pallas import tpu as pltpu
from jax.experimental.pallas import tpu_sc as plsc

B = 8192
N_T = 10
N_R = 16
E = 64
D = 64
H = 64
N_PAIR = 45

_PAIR_I = tuple(int(i) for i in range(N_T) for j in range(i + 1, N_T))
_PAIR_J = tuple(int(j) for i in range(N_T) for j in range(i + 1, N_T))

NT2 = N_T * N_T
NR2 = N_R * N_R
BB = 64
NW = 32
GPW = B // NW
CH = 16


NGRP = B // CH
GRP_PW = NGRP // NW


def _sc_hist(task_edge2, res_edge2):
    tt = task_edge2.reshape(NGRP, CH, 2 * E).transpose(0, 2, 1)
    rt = res_edge2.reshape(NGRP, CH, 2 * E).transpose(0, 2, 1)
    mesh = plsc.VectorSubcoreMesh(core_axis_name="c", subcore_axis_name="s")

    @functools.partial(
        pl.kernel,
        mesh=mesh,
        out_type=(
            jax.ShapeDtypeStruct((B * NT2,), jnp.float32),
            jax.ShapeDtypeStruct((B * NR2,), jnp.float32),
        ),
        scratch_types=[
            pltpu.VMEM((2 * E, CH), jnp.int32),
            pltpu.VMEM((2 * E, CH), jnp.int32),
            pltpu.VMEM((CH * NT2,), jnp.float32),
            pltpu.VMEM((CH * NR2,), jnp.float32),
        ],
        compiler_params=pltpu.CompilerParams(needs_layout_passes=False),
    )
    def k(te_hbm, re_hbm, at_hbm, ar_hbm, tev, rev, acct, accr):
        wid = lax.axis_index("s") * 2 + lax.axis_index("c")
        ones = jnp.full((16,), 1.0, jnp.float32)
        zeros = jnp.zeros((16,), jnp.float32)
        lane = lax.iota(jnp.int32, 16)

        def chunk(ci, carry):
            grp = wid * GRP_PW + ci
            pltpu.sync_copy(te_hbm.at[grp], tev)
            pltpu.sync_copy(re_hbm.at[grp], rev)
            for z in range(CH * NT2 // 16):
                acct[pl.ds(z * 16, 16)] = zeros
            for z in range(CH * NR2 // 16):
                accr[pl.ds(z * 16, 16)] = zeros
            for e in range(E):
                st = tev[e]
                dt = tev[e + E]
                plsc.addupdate_scatter(acct, [lane * NT2 + dt * N_T + st], ones)
                sr = rev[e]
                dr = rev[e + E]
                plsc.addupdate_scatter(accr, [lane * NR2 + dr * N_R + sr], ones)
            pltpu.sync_copy(acct, at_hbm.at[pl.ds(grp * CH * NT2, CH * NT2)])
            pltpu.sync_copy(accr, ar_hbm.at[pl.ds(grp * CH * NR2, CH * NR2)])
            return carry

        lax.fori_loop(0, GRP_PW, chunk, 0)

    return k(tt, rt)


def _layer_norm(z, g, b):
    m = z.mean(-1, keepdims=True)
    v = ((z - m) * (z - m)).mean(-1, keepdims=True)
    return (z - m) * lax.rsqrt(v + 1e-5) * g + b


def _tree_sum(terms):
    while len(terms) > 1:
        nxt = [terms[i] + terms[i + 1] for i in range(0, len(terms) - 1, 2)]
        if len(terms) % 2:
            nxt.append(terms[-1])
        terms = nxt
    return terms[0]


def _agg(An3, x3, n):
    return jax.lax.dot_general(An3, x3, (((2,), (1,)), ((0,), (0,))),
                               preferred_element_type=jnp.float32)


def _gnn_block(x3, n, f_in, A_ref,
               Wl1, bl1, Wr1, Wl2, bl2, Wr2, g1, b1, g2, b2, Wp, bp):
    A3 = A_ref[...].reshape(BB, n, n)
    cnt3 = A3.sum(-1, keepdims=True)
    An3 = A3 * (1.0 / jnp.maximum(cnt3, 1.0))

    xl = x3[:, :, 0:1] * Wl1[0:1, :]
    xr = x3[:, :, 0:1] * Wr1[0:1, :]
    for c in range(1, f_in):
        xl = xl + x3[:, :, c:c + 1] * Wl1[c:c + 1, :]
        xr = xr + x3[:, :, c:c + 1] * Wr1[c:c + 1, :]
    z1 = _agg(An3, xl, n) + bl1 + xr
    h1 = jnp.maximum(_layer_norm(z1, g1, b1), 0.0)

    h1f = h1.reshape(BB * n, H)
    hl = jnp.dot(h1f, Wl2, preferred_element_type=jnp.float32).reshape(BB, n, H)
    hr = jnp.dot(h1f, Wr2, preferred_element_type=jnp.float32).reshape(BB, n, H)
    z2 = _agg(An3, hl, n) + bl2 + hr
    h2 = jnp.maximum(_layer_norm(z2, g2, b2), 0.0)
    node = jnp.dot(h2.reshape(BB * n, H), Wp,
                   preferred_element_type=jnp.float32) + bp
    return node.reshape(BB, n, D)


def _fwd_kernel(tx_ref, at_ref, rx_ref, ar_ref, fb_ref,
                tWl1, tbl1, tWr1, tWl2, tbl2, tWr2, tg1, tb1, tg2, tb2, tWp, tbp,
                rWl1, rbl1, rWr1, rWl2, rbl2, rWr2, rg1, rb1, rg2, rb2, rWp, rbp,
                Wq, Wk, Wv, Wo, bo,
                wnoop_t, wnoop_r, wnoop_rb, wnoop_fb, bnoop,
                wval_t, wval_r, wval_rb, wval_fb, bval,
                wsplit, bsplit, Wm1, bm1, wm2, bm2, si_ref, sj_ref,
                logits_ref, value_ref):
    tx = tx_ref[...]
    rx = rx_ref[...]
    fb = fb_ref[...]

    task_node = _gnn_block(tx, N_T, 2, at_ref,
                           tWl1[...], tbl1[...], tWr1[...], tWl2[...], tbl2[...],
                           tWr2[...], tg1[...], tb1[...], tg2[...], tb2[...],
                           tWp[...], tbp[...])
    res_node = _gnn_block(rx, N_R, 3, ar_ref,
                          rWl1[...], rbl1[...], rWr1[...], rWl2[...], rbl2[...],
                          rWr2[...], rg1[...], rb1[...], rg2[...], rb2[...],
                          rWp[...], rbp[...])

    mask = (jnp.abs(tx).sum(-1, keepdims=True) > 0).astype(jnp.float32)
    task_node = task_node * mask

    tnf = task_node.reshape(BB * N_T, D)
    rnf = res_node.reshape(BB * N_R, D)
    Q = jnp.dot(tnf, Wq[...], preferred_element_type=jnp.float32).reshape(BB, N_T, D)
    K = jnp.dot(rnf, Wk[...], preferred_element_type=jnp.float32).reshape(BB, N_R, D)
    V = jnp.dot(rnf, Wv[...], preferred_element_type=jnp.float32).reshape(BB, N_R, D)

    inv_sqrt_d = 1.0 / (D ** 0.5)
    lg3 = jax.lax.dot_general(Q, K, (((2,), (2,)), ((0,), (0,))),
                              preferred_element_type=jnp.float32) * inv_sqrt_d
    mx = lg3.max(-1, keepdims=True)
    ex = jnp.exp(lg3 - mx)
    P3 = ex / ex.sum(-1, keepdims=True)
    ctx3 = jax.lax.dot_general(P3, V, (((2,), (1,)), ((0,), (0,))),
                               preferred_element_type=jnp.float32)
    te3 = task_node + jnp.dot(ctx3.reshape(BB * N_T, D), Wo[...],
                              preferred_element_type=jnp.float32
                              ).reshape(BB, N_T, D) + bo[...]

    n_real = jnp.maximum(mask.sum(axis=1), 1.0)
    tmean = te3.sum(axis=1) / n_real
    rmean = res_node.sum(axis=1) * (1.0 / N_R)
    rmin = rx.min(axis=1)
    rxmean = rx.sum(axis=1) * (1.0 / N_R)

    noop = ((tmean * wnoop_t[...]).sum(-1, keepdims=True)
            + (rmean * wnoop_r[...]).sum(-1, keepdims=True)
            + (rmin * wnoop_rb[...][:, 0:3]).sum(-1, keepdims=True)
            + (rxmean * wnoop_rb[...][:, 3:6]).sum(-1, keepdims=True)
            + (fb * wnoop_fb[...]).sum(-1, keepdims=True) + bnoop[...])
    val = ((tmean * wval_t[...]).sum(-1, keepdims=True)
           + (rmean * wval_r[...]).sum(-1, keepdims=True)
           + (rmin * wval_rb[...][:, 0:3]).sum(-1, keepdims=True)
           + (rxmean * wval_rb[...][:, 3:6]).sum(-1, keepdims=True)
           + (fb * wval_fb[...]).sum(-1, keepdims=True) + bval[...])
    value_ref[...] = val
    logits_ref[:, 0:1] = noop

    split = (te3 * wsplit[...][:, None, :]).sum(-1)
    logits_ref[:, 1:1 + N_T] = split + bsplit[...]

    W1 = Wm1[...]
    te_f = te3.reshape(BB * N_T, D)
    T1 = jnp.dot(te_f, W1[0:D, :], preferred_element_type=jnp.float32
                 ).reshape(BB, N_T, H)
    T2 = jnp.dot(te_f, W1[D:2 * D, :], preferred_element_type=jnp.float32
                 ).reshape(BB, N_T, H)
    wlq = W1[2 * D:2 * D + 1, :]
    SI = jnp.broadcast_to(si_ref[...][None], (BB, N_PAIR, N_T))
    SJ = jnp.broadcast_to(sj_ref[...][None], (BB, N_PAIR, N_T))

    def sel(S, x):
        return jax.lax.dot_general(S, x, (((2,), (1,)), ((0,), (0,))),
                                   preferred_element_type=jnp.float32)

    Pi3 = sel(SI, P3)
    Pj3 = sel(SJ, P3)
    lq3 = (Pi3 * Pj3).sum(-1, keepdims=True)
    T1p = sel(SI, T1)
    T2p = sel(SJ, T2)
    m3 = jnp.maximum(T1p + T2p + lq3 * wlq + bm1[...], 0.0)
    merge = (m3 * wm2[...]).sum(-1)
    logits_ref[:, 11:11 + N_PAIR] = merge + bm2[...]


def kernel(task_x, task_edge, res_x, res_edge, feedback, params):
    t = params['task']
    r = params['res']
    a = params['attn']
    h = params['heads']

    A_t, A_r = _sc_hist(task_edge.reshape(B, 2 * E), res_edge.reshape(B, 2 * E))
    A_t = A_t.reshape(B, NT2)
    A_r = A_r.reshape(B, NR2)

    def row(v):
        return v.reshape(1, -1)

    wnoop = h['Wnoop']
    wval = h['Wval']
    weights = (
        t['Wl1'], row(t['bl1']), t['Wr1'], t['Wl2'], row(t['bl2']), t['Wr2'],
        row(t['g1']), row(t['b1']), row(t['g2']), row(t['b2']), t['Wp'], row(t['bp']),
        r['Wl1'], row(r['bl1']), r['Wr1'], r['Wl2'], row(r['bl2']), r['Wr2'],
        row(r['g1']), row(r['b1']), row(r['g2']), row(r['b2']), r['Wp'], row(r['bp']),
        a['Wq'], a['Wk'], a['Wv'], a['Wo'], row(a['bo']),
        row(wnoop[0:D, 0]), row(wnoop[D:2 * D, 0]), row(wnoop[2 * D:2 * D + 6, 0][0:6]),
        row(wnoop[2 * D + 6:2 * D + 12, 0]), row(h['bnoop']),
        row(wval[0:D, 0]), row(wval[D:2 * D, 0]), row(wval[2 * D:2 * D + 6, 0][0:6]),
        row(wval[2 * D + 6:2 * D + 12, 0]), row(h['bval']),
        row(h['Wsplit'][:, 0]), row(h['bsplit']),
        h['Wm1'], row(h['bm1']), row(h['Wm2'][:, 0]), row(h['bm2']),
        jnp.asarray(np.eye(N_T, dtype=np.float32)[list(_PAIR_I)]),
        jnp.asarray(np.eye(N_T, dtype=np.float32)[list(_PAIR_J)]),
    )

    grid = (B // BB,)

    def bspec(shape, blk):
        nd = len(shape)
        return pl.BlockSpec(blk, lambda i: (i,) + (0,) * (nd - 1))

    def wspec(w):
        nd = w.ndim
        return pl.BlockSpec(w.shape, lambda i, _n=nd: (0,) * _n)

    in_specs = [
        bspec((B, N_T, 2), (BB, N_T, 2)),
        bspec((B, NT2), (BB, NT2)),
        bspec((B, N_R, 3), (BB, N_R, 3)),
        bspec((B, NR2), (BB, NR2)),
        bspec((B, 6), (BB, 6)),
    ] + [wspec(w) for w in weights]

    out_shape = (
        jax.ShapeDtypeStruct((B, 56), jnp.float32),
        jax.ShapeDtypeStruct((B, 1), jnp.float32),
    )
    out_specs = (
        pl.BlockSpec((BB, 56), lambda i: (i, 0)),
        pl.BlockSpec((BB, 1), lambda i: (i, 0)),
    )

    logits, value = pl.pallas_call(
        _fwd_kernel,
        grid=grid,
        in_specs=in_specs,
        out_specs=out_specs,
        out_shape=out_shape,
        compiler_params=pltpu.CompilerParams(
            dimension_semantics=("arbitrary",),
        ),
    )(task_x, A_t, res_x, A_r, feedback, *weights)
    return (logits, value)


# speedup vs baseline: 28.5593x; 1.0539x over previous
import functools
import numpy as np
import jax
import jax.numpy as jnp
from jax import lax
from jax.experimental import pallas as pl
from jax.experimental.pallas import tpu as pltpu
from jax.experimental.pallas import tpu_sc as plsc

B = 8192
N_T = 10
N_R = 16
E = 64
D = 64
H = 64
N_PAIR = 45

_PAIR_I = tuple(int(i) for i in range(N_T) for j in range(i + 1, N_T))
_PAIR_J = tuple(int(j) for i in range(N_T) for j in range(i + 1, N_T))

NT2 = N_T * N_T
NR2 = N_R * N_R
BB = 64
NW = 32
GPW = B // NW
CH = 16


NGRP = B // CH
GRP_PW = NGRP // NW


def _sc_hist(task_edge2, res_edge2):
    tt = task_edge2.reshape(NGRP, CH, 2 * E).transpose(0, 2, 1)
    rt = res_edge2.reshape(NGRP, CH, 2 * E).transpose(0, 2, 1)
    mesh = plsc.VectorSubcoreMesh(core_axis_name="c", subcore_axis_name="s")

    @functools.partial(
        pl.kernel,
        mesh=mesh,
        out_type=(
            jax.ShapeDtypeStruct((B * NT2,), jnp.float32),
            jax.ShapeDtypeStruct((B * NR2,), jnp.float32),
        ),
        scratch_types=[
            pltpu.VMEM((2 * E, CH), jnp.int32),
            pltpu.VMEM((2 * E, CH), jnp.int32),
            pltpu.VMEM((CH * NT2,), jnp.float32),
            pltpu.VMEM((CH * NR2,), jnp.float32),
        ],
        compiler_params=pltpu.CompilerParams(needs_layout_passes=False),
    )
    def k(te_hbm, re_hbm, at_hbm, ar_hbm, tev, rev, acct, accr):
        wid = lax.axis_index("s") * 2 + lax.axis_index("c")
        ones = jnp.full((16,), 1.0, jnp.float32)
        zeros = jnp.zeros((16,), jnp.float32)
        lane = lax.iota(jnp.int32, 16)

        def chunk(ci, carry):
            grp = wid * GRP_PW + ci
            pltpu.sync_copy(te_hbm.at[grp], tev)
            pltpu.sync_copy(re_hbm.at[grp], rev)
            for z in range(CH * NT2 // 16):
                acct[pl.ds(z * 16, 16)] = zeros
            for z in range(CH * NR2 // 16):
                accr[pl.ds(z * 16, 16)] = zeros
            for e in range(E):
                st = tev[e]
                dt = tev[e + E]
                plsc.addupdate_scatter(acct, [lane * NT2 + dt * N_T + st], ones)
                sr = rev[e]
                dr = rev[e + E]
                plsc.addupdate_scatter(accr, [lane * NR2 + dr * N_R + sr], ones)
            pltpu.sync_copy(acct, at_hbm.at[pl.ds(grp * CH * NT2, CH * NT2)])
            pltpu.sync_copy(accr, ar_hbm.at[pl.ds(grp * CH * NR2, CH * NR2)])
            return carry

        lax.fori_loop(0, GRP_PW, chunk, 0)

    return k(tt, rt)


def _layer_norm(z, g, b):
    m = z.mean(-1, keepdims=True)
    v = ((z - m) * (z - m)).mean(-1, keepdims=True)
    return (z - m) * lax.rsqrt(v + 1e-5) * g + b


def _tree_sum(terms):
    while len(terms) > 1:
        nxt = [terms[i] + terms[i + 1] for i in range(0, len(terms) - 1, 2)]
        if len(terms) % 2:
            nxt.append(terms[-1])
        terms = nxt
    return terms[0]


def _agg(An3, x3, n):
    return jax.lax.dot_general(An3, x3, (((2,), (1,)), ((0,), (0,))),
                               preferred_element_type=jnp.float32)


def _gnn_block(x3, n, f_in, A_ref,
               Wl1, bl1, Wr1, Wl2, bl2, Wr2, g1, b1, g2, b2, Wp, bp):
    A3 = A_ref[...].reshape(BB, n, n)
    cnt3 = A3.sum(-1, keepdims=True)
    An3 = A3 * (1.0 / jnp.maximum(cnt3, 1.0))

    xf = x3.reshape(BB * n, f_in)
    xl = jnp.dot(xf, Wl1, preferred_element_type=jnp.float32).reshape(BB, n, H)
    xr = jnp.dot(xf, Wr1, preferred_element_type=jnp.float32).reshape(BB, n, H)
    z1 = _agg(An3, xl, n) + bl1 + xr
    h1 = jnp.maximum(_layer_norm(z1, g1, b1), 0.0)

    h1f = h1.reshape(BB * n, H)
    hl = jnp.dot(h1f, Wl2, preferred_element_type=jnp.float32).reshape(BB, n, H)
    hr = jnp.dot(h1f, Wr2, preferred_element_type=jnp.float32).reshape(BB, n, H)
    z2 = _agg(An3, hl, n) + bl2 + hr
    h2 = jnp.maximum(_layer_norm(z2, g2, b2), 0.0)
    node = jnp.dot(h2.reshape(BB * n, H), Wp,
                   preferred_element_type=jnp.float32) + bp
    return node.reshape(BB, n, D)


def _fwd_kernel(tx_ref, at_ref, rx_ref, ar_ref, fb_ref,
                tWl1, tbl1, tWr1, tWl2, tbl2, tWr2, tg1, tb1, tg2, tb2, tWp, tbp,
                rWl1, rbl1, rWr1, rWl2, rbl2, rWr2, rg1, rb1, rg2, rb2, rWp, rbp,
                Wq, Wk, Wv, Wo, bo,
                wnoop_t, wnoop_r, wnoop_rb, wnoop_fb, bnoop,
                wval_t, wval_r, wval_rb, wval_fb, bval,
                wsplit, bsplit, Wm1, bm1, wm2, bm2, si_ref, sj_ref,
                logits_ref, value_ref):
    tx = tx_ref[...]
    rx = rx_ref[...]
    fb = fb_ref[...]

    task_node = _gnn_block(tx, N_T, 2, at_ref,
                           tWl1[...], tbl1[...], tWr1[...], tWl2[...], tbl2[...],
                           tWr2[...], tg1[...], tb1[...], tg2[...], tb2[...],
                           tWp[...], tbp[...])
    res_node = _gnn_block(rx, N_R, 3, ar_ref,
                          rWl1[...], rbl1[...], rWr1[...], rWl2[...], rbl2[...],
                          rWr2[...], rg1[...], rb1[...], rg2[...], rb2[...],
                          rWp[...], rbp[...])

    mask = (jnp.abs(tx).sum(-1, keepdims=True) > 0).astype(jnp.float32)
    task_node = task_node * mask

    tnf = task_node.reshape(BB * N_T, D)
    rnf = res_node.reshape(BB * N_R, D)
    Q = jnp.dot(tnf, Wq[...], preferred_element_type=jnp.float32).reshape(BB, N_T, D)
    K = jnp.dot(rnf, Wk[...], preferred_element_type=jnp.float32).reshape(BB, N_R, D)
    V = jnp.dot(rnf, Wv[...], preferred_element_type=jnp.float32).reshape(BB, N_R, D)

    inv_sqrt_d = 1.0 / (D ** 0.5)
    lg3 = jax.lax.dot_general(Q, K, (((2,), (2,)), ((0,), (0,))),
                              preferred_element_type=jnp.float32) * inv_sqrt_d
    mx = lg3.max(-1, keepdims=True)
    ex = jnp.exp(lg3 - mx)
    P3 = ex / ex.sum(-1, keepdims=True)
    ctx3 = jax.lax.dot_general(P3, V, (((2,), (1,)), ((0,), (0,))),
                               preferred_element_type=jnp.float32)
    te3 = task_node + jnp.dot(ctx3.reshape(BB * N_T, D), Wo[...],
                              preferred_element_type=jnp.float32
                              ).reshape(BB, N_T, D) + bo[...]

    n_real = jnp.maximum(mask.sum(axis=1), 1.0)
    tmean = te3.sum(axis=1) / n_real
    rmean = res_node.sum(axis=1) * (1.0 / N_R)
    rmin = rx.min(axis=1)
    rxmean = rx.sum(axis=1) * (1.0 / N_R)

    noop = ((tmean * wnoop_t[...]).sum(-1, keepdims=True)
            + (rmean * wnoop_r[...]).sum(-1, keepdims=True)
            + (rmin * wnoop_rb[...][:, 0:3]).sum(-1, keepdims=True)
            + (rxmean * wnoop_rb[...][:, 3:6]).sum(-1, keepdims=True)
            + (fb * wnoop_fb[...]).sum(-1, keepdims=True) + bnoop[...])
    val = ((tmean * wval_t[...]).sum(-1, keepdims=True)
           + (rmean * wval_r[...]).sum(-1, keepdims=True)
           + (rmin * wval_rb[...][:, 0:3]).sum(-1, keepdims=True)
           + (rxmean * wval_rb[...][:, 3:6]).sum(-1, keepdims=True)
           + (fb * wval_fb[...]).sum(-1, keepdims=True) + bval[...])
    value_ref[...] = val
    logits_ref[:, 0:1] = noop

    split = (te3 * wsplit[...][:, None, :]).sum(-1)
    logits_ref[:, 1:1 + N_T] = split + bsplit[...]

    W1 = Wm1[...]
    te_f = te3.reshape(BB * N_T, D)
    T1 = jnp.dot(te_f, W1[0:D, :], preferred_element_type=jnp.float32
                 ).reshape(BB, N_T, H)
    T2 = jnp.dot(te_f, W1[D:2 * D, :], preferred_element_type=jnp.float32
                 ).reshape(BB, N_T, H)
    wlq = W1[2 * D:2 * D + 1, :]
    SI = jnp.broadcast_to(si_ref[...][None], (BB, N_PAIR, N_T))
    SJ = jnp.broadcast_to(sj_ref[...][None], (BB, N_PAIR, N_T))

    def sel(S, x):
        return jax.lax.dot_general(S, x, (((2,), (1,)), ((0,), (0,))),
                                   preferred_element_type=jnp.float32)

    Pi3 = sel(SI, P3)
    Pj3 = sel(SJ, P3)
    lq3 = (Pi3 * Pj3).sum(-1, keepdims=True)
    T1p = sel(SI, T1)
    T2p = sel(SJ, T2)
    m3 = jnp.maximum(T1p + T2p + lq3 * wlq + bm1[...], 0.0)
    merge = (m3 * wm2[...]).sum(-1)
    logits_ref[:, 11:11 + N_PAIR] = merge + bm2[...]


def kernel(task_x, task_edge, res_x, res_edge, feedback, params):
    t = params['task']
    r = params['res']
    a = params['attn']
    h = params['heads']

    A_t, A_r = _sc_hist(task_edge.reshape(B, 2 * E), res_edge.reshape(B, 2 * E))
    A_t = A_t.reshape(B, NT2)
    A_r = A_r.reshape(B, NR2)

    def row(v):
        return v.reshape(1, -1)

    wnoop = h['Wnoop']
    wval = h['Wval']
    weights = (
        t['Wl1'], row(t['bl1']), t['Wr1'], t['Wl2'], row(t['bl2']), t['Wr2'],
        row(t['g1']), row(t['b1']), row(t['g2']), row(t['b2']), t['Wp'], row(t['bp']),
        r['Wl1'], row(r['bl1']), r['Wr1'], r['Wl2'], row(r['bl2']), r['Wr2'],
        row(r['g1']), row(r['b1']), row(r['g2']), row(r['b2']), r['Wp'], row(r['bp']),
        a['Wq'], a['Wk'], a['Wv'], a['Wo'], row(a['bo']),
        row(wnoop[0:D, 0]), row(wnoop[D:2 * D, 0]), row(wnoop[2 * D:2 * D + 6, 0][0:6]),
        row(wnoop[2 * D + 6:2 * D + 12, 0]), row(h['bnoop']),
        row(wval[0:D, 0]), row(wval[D:2 * D, 0]), row(wval[2 * D:2 * D + 6, 0][0:6]),
        row(wval[2 * D + 6:2 * D + 12, 0]), row(h['bval']),
        row(h['Wsplit'][:, 0]), row(h['bsplit']),
        h['Wm1'], row(h['bm1']), row(h['Wm2'][:, 0]), row(h['bm2']),
        jnp.asarray(np.eye(N_T, dtype=np.float32)[list(_PAIR_I)]),
        jnp.asarray(np.eye(N_T, dtype=np.float32)[list(_PAIR_J)]),
    )

    grid = (B // BB,)

    def bspec(shape, blk):
        nd = len(shape)
        return pl.BlockSpec(blk, lambda i: (i,) + (0,) * (nd - 1))

    def wspec(w):
        nd = w.ndim
        return pl.BlockSpec(w.shape, lambda i, _n=nd: (0,) * _n)

    in_specs = [
        bspec((B, N_T, 2), (BB, N_T, 2)),
        bspec((B, NT2), (BB, NT2)),
        bspec((B, N_R, 3), (BB, N_R, 3)),
        bspec((B, NR2), (BB, NR2)),
        bspec((B, 6), (BB, 6)),
    ] + [wspec(w) for w in weights]

    out_shape = (
        jax.ShapeDtypeStruct((B, 56), jnp.float32),
        jax.ShapeDtypeStruct((B, 1), jnp.float32),
    )
    out_specs = (
        pl.BlockSpec((BB, 56), lambda i: (i, 0)),
        pl.BlockSpec((BB, 1), lambda i: (i, 0)),
    )

    logits, value = pl.pallas_call(
        _fwd_kernel,
        grid=grid,
        in_specs=in_specs,
        out_specs=out_specs,
        out_shape=out_shape,
        compiler_params=pltpu.CompilerParams(
            dimension_semantics=("arbitrary",),
        ),
    )(task_x, A_t, res_x, A_r, feedback, *weights)
    return (logits, value)
